# Initial kernel scaffold; baseline (speedup 1.0000x reference)
import jax
import jax.numpy as jnp
from jax.experimental import pallas as pl


def kernel(x, edge_index, edge_attr, batch, atom_emb, bond_emb, conv_A, conv_S, conv_b, bn_g, bn_b, pool_A, pool_S, pool_b, pool_rW, pool_rb, dn_A1, dn_S1, dn_b1, dn_g1, dn_bb1, dn_A2, dn_S2, dn_b2, dn_g2, dn_bb2, dn_A3, dn_S3, dn_b3, dn_rW, dn_rb):
    raise NotImplementedError("write your pallas kernel here")


# speedup vs baseline: 8.4730x; 8.4730x over previous
import functools

import jax
import jax.numpy as jnp
from jax import lax
from jax.experimental import pallas as pl
from jax.experimental.pallas import tpu as pltpu
from jax.experimental.pallas import tpu_sc as plsc

P = 4
N = 10000
E = 320000
G = 128
D = 196
DP = 208
WA = 128
WB = 80
CH = 128
NCH = E // CH
NC = 2
NS = 16
NW = NC * NS
RPT8 = 624
RPTL = N - RPT8 * (NS - 1)

_f32 = jnp.float32

def _phm_w(A, S):
    p, ic, oc = S.shape
    return jnp.einsum('nij,nab->iajb', A, S).reshape(p * ic, p * oc)


def _rows_copy(s, copy_fn):
    @pl.when(s < NS - 1)
    def _():
        copy_fn(s * RPT8, RPT8)

    @pl.when(s == NS - 1)
    def _():
        copy_fn(RPT8 * (NS - 1), RPTL)


def _stats_body(dst_hbm, vals_hbm, z8_hbm, out_hbm, idx_d, vbuf, st_sh):
    c = lax.axis_index("c")
    s = lax.axis_index("s")
    w = s * NC + c
    _rows_copy(s, lambda r0, nr: pltpu.sync_copy(
        z8_hbm.at[pl.ds(r0, nr)], st_sh.at[pl.ds(r0, nr)]))
    plsc.subcore_barrier()
    nch = NCH // NW
    rem = NCH % NW
    myn = nch + jnp.where(w < rem, 1, 0)
    cb = w * nch + jnp.minimum(w, rem)

    def body(i, carry):
        b = (cb + i) * CH
        pltpu.sync_copy(dst_hbm.at[pl.ds(b, CH)], idx_d)
        pltpu.sync_copy(vals_hbm.at[pl.ds(b, CH)], vbuf)
        pltpu.sync_copy(vbuf, st_sh.at[idx_d], add=True)
        return carry

    lax.fori_loop(0, myn, body, 0)
    plsc.subcore_barrier()
    _rows_copy(s, lambda r0, nr: pltpu.sync_copy(
        st_sh.at[pl.ds(r0, nr)], out_hbm.at[c, pl.ds(r0, nr)]))


def _segsum_body(table_hbm, src_hbm, dst_hbm, zero_hbm, out_hbm,
                 idx_s, idx_d, rows, acc_sh, gsem):
    c = lax.axis_index("c")
    s = lax.axis_index("s")
    w = s * NC + c

    @pl.when(c == 0)
    def _():
        _rows_copy(s, lambda r0, nr: pltpu.sync_copy(
            table_hbm.at[pl.ds(r0, nr)], acc_sh.at[pl.ds(r0, nr)]))

    @pl.when(c == 1)
    def _():
        _rows_copy(s, lambda r0, nr: pltpu.sync_copy(
            zero_hbm.at[pl.ds(r0, nr)], acc_sh.at[pl.ds(r0, nr)]))

    plsc.subcore_barrier()
    nch = NCH // NW
    rem = NCH % NW
    myn = nch + jnp.where(w < rem, 1, 0)
    cb = w * nch + jnp.minimum(w, rem)

    def body(i, carry):
        b = (cb + i) * CH
        pltpu.sync_copy(src_hbm.at[pl.ds(b, CH)], idx_s)
        pltpu.sync_copy(dst_hbm.at[pl.ds(b, CH)], idx_d)
        pltpu.async_copy(table_hbm.at[idx_s], rows, gsem).wait()
        pltpu.sync_copy(rows, acc_sh.at[idx_d], add=True)
        return carry

    lax.fori_loop(0, myn, body, 0)
    plsc.subcore_barrier()
    _rows_copy(s, lambda r0, nr: pltpu.sync_copy(
        acc_sh.at[pl.ds(r0, nr)], out_hbm.at[c, pl.ds(r0, nr)]))


@functools.cache
def _sc_kernels():
    mesh = plsc.VectorSubcoreMesh(core_axis_name="c", subcore_axis_name="s",
                                  num_cores=NC, num_subcores=NS)
    stats = pl.kernel(
        _stats_body, mesh=mesh,
        compiler_params=pltpu.CompilerParams(use_tc_tiling_on_sc=False),
        out_type=jax.ShapeDtypeStruct((NC, N, 8), _f32),
        scratch_types=[
            pltpu.VMEM((CH,), jnp.int32),
            pltpu.VMEM((CH, 8), _f32),
            pltpu.VMEM_SHARED((N, 8), _f32),
        ],
    )

    def seg(width):
        return pl.kernel(
            _segsum_body, mesh=mesh,
            compiler_params=pltpu.CompilerParams(use_tc_tiling_on_sc=False),
            out_type=jax.ShapeDtypeStruct((NC, N, width), _f32),
            scratch_types=[
                pltpu.VMEM((CH,), jnp.int32),
                pltpu.VMEM((CH,), jnp.int32),
                pltpu.VMEM((CH, width), _f32),
                pltpu.VMEM_SHARED((N, width), _f32),
                pltpu.SemaphoreType.DMA,
            ],
        )

    return stats, seg(WA), seg(WB)


def _stats_sc(dst, vals, z8):
    return _sc_kernels()[0](dst, vals, z8)


def _segsum_sc(table_a, table_b, src, dst, zeros_a, zeros_b):
    _, seg_a, seg_b = _sc_kernels()
    return seg_a(table_a, src, dst, zeros_a), seg_b(table_b, src, dst, zeros_b)


def _atom_body(xf_ref, c0_ref, da_ref, out_a_ref, out_b_ref):
    full = c0_ref[...] + jnp.dot(
        xf_ref[...], da_ref[...], preferred_element_type=_f32, precision=lax.Precision.HIGHEST)
    out_a_ref[...] = full[:, :WA]
    out_b_ref[...] = full[:, WA:]


def _make_atom_tc(interpret=False):
    return pl.pallas_call(
        _atom_body,
        interpret=interpret,
        grid=(NB,),
        in_specs=[
            pl.BlockSpec((BR, 16), lambda i: (i, 0)),
            pl.BlockSpec((1, DP), lambda i: (0, 0)),
            pl.BlockSpec((16, DP), lambda i: (0, 0)),
        ],
        out_specs=(pl.BlockSpec((BR, WA), lambda i: (i, 0)),
                   pl.BlockSpec((BR, WB), lambda i: (i, 0))),
        out_shape=(jax.ShapeDtypeStruct((N, WA), _f32),
                   jax.ShapeDtypeStruct((N, WB), _f32)))


BR = 1000
NB = N // BR


def _dot(a, b):
    return jnp.dot(a, b, preferred_element_type=_f32,
                   precision=lax.Precision.HIGHEST)


def _dotd(a, b):
    return jnp.dot(a, b, preferred_element_type=_f32)


def _bn(y, g, b):
    m = jnp.mean(y, axis=0, keepdims=True)
    v = jnp.mean((y - m) * (y - m), axis=0, keepdims=True)
    return (y - m) / jnp.sqrt(v + 1e-5) * g + b


def _layer_mm_body(parts_a_ref, parts_b_ref, stats_ref, wb_ref, wl_ref,
                   cb_ref, y_ref, sums_ref, accm, acc2):
    i = pl.program_id(0)
    nbr = jnp.concatenate(
        [parts_a_ref[0] + parts_a_ref[1],
         parts_b_ref[0] + parts_b_ref[1]], axis=1)
    st = stats_ref[0] + stats_ref[1]
    agg = nbr + _dot(st, wb_ref[...])
    y = _dotd(agg, wl_ref[...]) + cb_ref[...]
    y_ref[...] = y

    @pl.when(i == 0)
    def _():
        accm[...] = jnp.zeros_like(accm)
        acc2[...] = jnp.zeros_like(acc2)

    mb = jnp.mean(y, axis=0, keepdims=True)
    d = y - mb
    accm[pl.ds(i, 1), :] = mb
    acc2[pl.ds(i, 1), :] = jnp.sum(d * d, axis=0, keepdims=True)

    @pl.when(i == NB - 1)
    def _():
        rmask = (lax.broadcasted_iota(jnp.int32, (16, 1), 0) < NB
                 ).astype(_f32)
        means = accm[...] * rmask
        m = jnp.sum(means, axis=0, keepdims=True) * (1.0 / NB)
        dev = (accm[...] - m) * rmask
        v = (jnp.sum(acc2[...] * rmask, axis=0, keepdims=True)
             + BR * jnp.sum(dev * dev, axis=0, keepdims=True)) * (1.0 / N)
        sums_ref[...] = jnp.concatenate(
            [m, v, jnp.zeros((6, DP), _f32)], axis=0)


def _make_layer_mm_tc(interpret=False):
    return pl.pallas_call(
    _layer_mm_body,
    interpret=interpret,
    grid=(NB,),
    in_specs=[
        pl.BlockSpec((NC, BR, WA), lambda i: (0, i, 0)),
        pl.BlockSpec((NC, BR, WB), lambda i: (0, i, 0)),
        pl.BlockSpec((NC, BR, 8), lambda i: (0, i, 0)),
        pl.BlockSpec((8, DP), lambda i: (0, 0)),
        pl.BlockSpec((DP, DP), lambda i: (0, 0)),
        pl.BlockSpec((1, DP), lambda i: (0, 0)),
    ],
    out_specs=(pl.BlockSpec((BR, DP), lambda i: (i, 0)),
               pl.BlockSpec((8, DP), lambda i: (0, 0))),
    out_shape=(jax.ShapeDtypeStruct((N, DP), _f32),
               jax.ShapeDtypeStruct((8, DP), _f32)),
    scratch_shapes=[pltpu.VMEM((16, DP), _f32), pltpu.VMEM((16, DP), _f32)],
    )


def _layer_norm_body(y_ref, sums_ref, g_ref, b_ref, atom_a_ref, atom_b_ref,
                     out_a_ref, out_b_ref):
    m = sums_ref[0:1, :]
    v = sums_ref[1:2, :]
    yn = (y_ref[...] - m) / jnp.sqrt(v + 1e-5) * g_ref[...] + b_ref[...]
    out = jnp.maximum(yn, 0.0)
    out_a_ref[...] = out[:, :WA] + atom_a_ref[...]
    out_b_ref[...] = out[:, WA:] + atom_b_ref[...]


def _make_layer_norm_tc(interpret=False):
    return pl.pallas_call(
    _layer_norm_body,
    interpret=interpret,
    grid=(NB,),
    in_specs=[
        pl.BlockSpec((BR, DP), lambda i: (i, 0)),
        pl.BlockSpec((8, DP), lambda i: (0, 0)),
        pl.BlockSpec((1, DP), lambda i: (0, 0)),
        pl.BlockSpec((1, DP), lambda i: (0, 0)),
        pl.BlockSpec((BR, WA), lambda i: (i, 0)),
        pl.BlockSpec((BR, WB), lambda i: (i, 0)),
    ],
    out_specs=(pl.BlockSpec((BR, WA), lambda i: (i, 0)),
               pl.BlockSpec((BR, WB), lambda i: (i, 0))),
    out_shape=(jax.ShapeDtypeStruct((N, WA), _f32),
               jax.ShapeDtypeStruct((N, WB), _f32)),
    )


def _layer_tc(parts_a, parts_b, stats, atom_a, atom_b, wbl, wll, cbl, gl, bl):
    y, sums = _layer_mm_tc(parts_a, parts_b, stats, wbl, wll, cbl)
    return _layer_norm_tc(y, sums, gl, bl, atom_a, atom_b)


def _pool_body(hx_a_ref, hx_b_ref, batch_ref, wp_ref, pb_ref, prw_ref, prb_ref,
               w1_ref, b1_ref, g1_ref, bb1_ref,
               w2_ref, b2_ref, g2_ref, bb2_ref,
               w3_ref, b3_ref, rw_ref, rb_ref, out_ref, acc):
    i = pl.program_id(0)
    hx = jnp.concatenate([hx_a_ref[...], hx_b_ref[...]], axis=1)
    att = _dotd(hx, wp_ref[...]) + pb_ref[...]
    sc = jax.nn.sigmoid(_dotd(att, prw_ref[...]) + prb_ref[...])
    sh = hx * sc
    gi = lax.broadcasted_iota(jnp.int32, (G, BR), 0)
    oh = (batch_ref[0] == gi).astype(_f32)

    @pl.when(i == 0)
    def _():
        acc[...] = jnp.zeros_like(acc)

    acc[...] += _dot(oh, sh)

    @pl.when(i == NB - 1)
    def _():
        pooled = acc[...]
        z = _dotd(pooled, w1_ref[...]) + b1_ref[...]
        z = jnp.maximum(_bn(z, g1_ref[...], bb1_ref[...]), 0.0)
        z = _dotd(z, w2_ref[...]) + b2_ref[...]
        z = jnp.maximum(_bn(z, g2_ref[...], bb2_ref[...]), 0.0)
        z = _dotd(z, w3_ref[...]) + b3_ref[...]
        out_ref[...] = _dotd(z, rw_ref[...]) + rb_ref[...]


def _w_spec(shape):
    return pl.BlockSpec(shape, lambda i: tuple(0 for _ in shape))


def _make_pool_tc(interpret=False):
    return pl.pallas_call(
    _pool_body,
    interpret=interpret,
    grid=(NB,),
    in_specs=[
        pl.BlockSpec((BR, WA), lambda i: (i, 0)),
        pl.BlockSpec((BR, WB), lambda i: (i, 0)),
        pl.BlockSpec((1, 1, BR), lambda i: (i, 0, 0)),
        _w_spec((DP, DP)), _w_spec((1, DP)), _w_spec((DP, 1)), _w_spec((1, 1)),
        _w_spec((DP, 256)), _w_spec((1, 256)), _w_spec((1, 256)),
        _w_spec((1, 256)),
        _w_spec((256, 128)), _w_spec((1, 128)), _w_spec((1, 128)),
        _w_spec((1, 128)),
        _w_spec((128, P)), _w_spec((1, P)), _w_spec((P, 1)), _w_spec((1, 1)),
    ],
    out_specs=pl.BlockSpec((G, 1), lambda i: (0, 0)),
    out_shape=jax.ShapeDtypeStruct((G, 1), _f32),
    scratch_shapes=[pltpu.VMEM((G, DP), _f32)],
    )


_atom_tc = _make_atom_tc()
_layer_mm_tc = _make_layer_mm_tc()
_layer_norm_tc = _make_layer_norm_tc()
_pool_tc = _make_pool_tc()


def kernel(x, edge_index, edge_attr, batch, atom_emb, bond_emb, conv_A,
           conv_S, conv_b, bn_g, bn_b, pool_A, pool_S, pool_b, pool_rW,
           pool_rb, dn_A1, dn_S1, dn_b1, dn_g1, dn_bb1, dn_A2, dn_S2, dn_b2,
           dn_g2, dn_bb2, dn_A3, dn_S3, dn_b3, dn_rW, dn_rb):
    pad = DP - D
    src = edge_index[0].astype(jnp.int32)
    dst = edge_index[1].astype(jnp.int32)

    vals = jnp.concatenate(
        [jnp.ones((E, 1), _f32), edge_attr.astype(_f32),
         jnp.zeros((E, 4), _f32)], axis=1)

    xf = jnp.pad(x.astype(_f32), ((0, 0), (0, 16 - x.shape[1])))
    da = jnp.pad(atom_emb[:, 1, :] - atom_emb[:, 0, :],
                 ((0, 16 - atom_emb.shape[0]), (0, pad)))
    c0 = jnp.pad(atom_emb[:, 0, :].sum(0), (0, pad)).reshape(1, DP)

    b0 = bond_emb[:, :, 0, :].sum(1)
    dB = bond_emb[:, :, 1, :] - bond_emb[:, :, 0, :]
    wb = jnp.concatenate(
        [b0[:, None, :], dB, jnp.zeros((3, 4, D), _f32)], axis=1)
    wb = jnp.pad(wb, ((0, 0), (0, 0), (0, pad)))

    wl = jnp.stack([_phm_w(conv_A[l], conv_S[l]) for l in range(3)])
    wl = jnp.pad(wl, ((0, 0), (0, pad), (0, pad)))
    cb = jnp.pad(conv_b, ((0, 0), (0, pad)))
    gg = jnp.pad(bn_g, ((0, 0), (0, pad)))
    bb = jnp.pad(bn_b, ((0, 0), (0, pad)))

    zeros_a = jnp.zeros((N, WA), _f32)
    zeros_b = jnp.zeros((N, WB), _f32)
    z8 = jnp.zeros((N, 8), _f32)

    wp = jnp.pad(_phm_w(pool_A, pool_S), ((0, pad), (0, pad)))
    pb = jnp.pad(pool_b, (0, pad)).reshape(1, DP)
    prw = jnp.pad(pool_rW, ((0, pad), (0, 0)))
    prb = pool_rb.reshape(1, 1)
    w1 = jnp.pad(_phm_w(dn_A1, dn_S1), ((0, pad), (0, 0)))
    w2 = _phm_w(dn_A2, dn_S2)
    w3 = _phm_w(dn_A3, dn_S3)

    atom_a, atom_b = _atom_tc(xf, c0, da)
    stats = _stats_sc(dst, vals, z8)

    hx_a, hx_b = atom_a, atom_b
    for l in range(3):
        nbr_a, nbr_b = _segsum_sc(hx_a, hx_b, src, dst, zeros_a, zeros_b)
        hx_a, hx_b = _layer_tc(nbr_a, nbr_b, stats, atom_a, atom_b,
                               wb[l], wl[l],
                               cb[l].reshape(1, DP), gg[l].reshape(1, DP),
                               bb[l].reshape(1, DP))

    return _pool_tc(hx_a, hx_b, batch.astype(jnp.int32).reshape(NB, 1, BR),
                    wp, pb, prw, prb,
                    w1, dn_b1.reshape(1, -1), dn_g1.reshape(1, -1),
                    dn_bb1.reshape(1, -1),
                    w2, dn_b2.reshape(1, -1), dn_g2.reshape(1, -1),
                    dn_bb2.reshape(1, -1),
                    w3, dn_b3.reshape(1, -1), dn_rW, dn_rb.reshape(1, 1))


# speedup vs baseline: 8.4735x; 1.0001x over previous
import functools

import jax
import jax.numpy as jnp
from jax import lax
from jax.experimental import pallas as pl
from jax.experimental.pallas import tpu as pltpu
from jax.experimental.pallas import tpu_sc as plsc

P = 4
N = 10000
E = 320000
G = 128
D = 196
DP = 208
WA = 128
WB = 80
CH = 128
NCH = E // CH
NC = 2
NS = 16
NW = NC * NS
RPT8 = 624
RPTL = N - RPT8 * (NS - 1)

_f32 = jnp.float32

def _phm_w(A, S):
    p, ic, oc = S.shape
    return jnp.einsum('nij,nab->iajb', A, S).reshape(p * ic, p * oc)


def _rows_copy(s, copy_fn):
    @pl.when(s < NS - 1)
    def _():
        copy_fn(s * RPT8, RPT8)

    @pl.when(s == NS - 1)
    def _():
        copy_fn(RPT8 * (NS - 1), RPTL)


def _stats_body(dst_hbm, vals_hbm, z8_hbm, out_hbm, idx_d, vbuf, st_sh):
    c = lax.axis_index("c")
    s = lax.axis_index("s")
    w = s * NC + c
    _rows_copy(s, lambda r0, nr: pltpu.sync_copy(
        z8_hbm.at[pl.ds(r0, nr)], st_sh.at[pl.ds(r0, nr)]))
    plsc.subcore_barrier()
    nch = NCH // NW
    rem = NCH % NW
    myn = nch + jnp.where(w < rem, 1, 0)
    cb = w * nch + jnp.minimum(w, rem)

    def body(i, carry):
        b = (cb + i) * CH
        pltpu.sync_copy(dst_hbm.at[pl.ds(b, CH)], idx_d)
        pltpu.sync_copy(vals_hbm.at[pl.ds(b, CH)], vbuf)
        pltpu.sync_copy(vbuf, st_sh.at[idx_d], add=True)
        return carry

    lax.fori_loop(0, myn, body, 0)
    plsc.subcore_barrier()
    _rows_copy(s, lambda r0, nr: pltpu.sync_copy(
        st_sh.at[pl.ds(r0, nr)], out_hbm.at[c, pl.ds(r0, nr)]))


def _segsum_body(table_hbm, src_hbm, dst_hbm, zero_hbm, out_hbm,
                 idx_s, idx_d, rows, acc_sh, gsem):
    c = lax.axis_index("c")
    s = lax.axis_index("s")
    w = s * NC + c

    @pl.when(c == 0)
    def _():
        _rows_copy(s, lambda r0, nr: pltpu.sync_copy(
            table_hbm.at[pl.ds(r0, nr)], acc_sh.at[pl.ds(r0, nr)]))

    @pl.when(c == 1)
    def _():
        _rows_copy(s, lambda r0, nr: pltpu.sync_copy(
            zero_hbm.at[pl.ds(r0, nr)], acc_sh.at[pl.ds(r0, nr)]))

    plsc.subcore_barrier()
    nch = NCH // NW
    rem = NCH % NW
    myn = nch + jnp.where(w < rem, 1, 0)
    cb = w * nch + jnp.minimum(w, rem)

    def body(i, carry):
        b = (cb + i) * CH
        pltpu.sync_copy(src_hbm.at[pl.ds(b, CH)], idx_s)
        pltpu.sync_copy(dst_hbm.at[pl.ds(b, CH)], idx_d)
        pltpu.async_copy(table_hbm.at[idx_s], rows, gsem).wait()
        pltpu.sync_copy(rows, acc_sh.at[idx_d], add=True)
        return carry

    lax.fori_loop(0, myn, body, 0)
    plsc.subcore_barrier()
    _rows_copy(s, lambda r0, nr: pltpu.sync_copy(
        acc_sh.at[pl.ds(r0, nr)], out_hbm.at[c, pl.ds(r0, nr)]))


@functools.cache
def _sc_kernels():
    mesh = plsc.VectorSubcoreMesh(core_axis_name="c", subcore_axis_name="s",
                                  num_cores=NC, num_subcores=NS)
    stats = pl.kernel(
        _stats_body, mesh=mesh,
        compiler_params=pltpu.CompilerParams(use_tc_tiling_on_sc=False),
        out_type=jax.ShapeDtypeStruct((NC, N, 8), _f32),
        scratch_types=[
            pltpu.VMEM((CH,), jnp.int32),
            pltpu.VMEM((CH, 8), _f32),
            pltpu.VMEM_SHARED((N, 8), _f32),
        ],
    )

    def seg(width):
        return pl.kernel(
            _segsum_body, mesh=mesh,
            compiler_params=pltpu.CompilerParams(use_tc_tiling_on_sc=False),
            out_type=jax.ShapeDtypeStruct((NC, N, width), _f32),
            scratch_types=[
                pltpu.VMEM((CH,), jnp.int32),
                pltpu.VMEM((CH,), jnp.int32),
                pltpu.VMEM((CH, width), _f32),
                pltpu.VMEM_SHARED((N, width), _f32),
                pltpu.SemaphoreType.DMA,
            ],
        )

    return stats, seg(WA), seg(WB)


def _stats_sc(dst, vals, z8):
    return _sc_kernels()[0](dst, vals, z8)


def _segsum_sc(table_a, table_b, src, dst, zeros_a, zeros_b):
    _, seg_a, seg_b = _sc_kernels()
    return seg_a(table_a, src, dst, zeros_a), seg_b(table_b, src, dst, zeros_b)


def _atom_body(xi_ref, t0_ref, t1_ref, out_a_ref, out_b_ref):
    full = jnp.zeros((xi_ref.shape[0], DP), _f32)
    for f in range(9):
        sel = (xi_ref[:, f:f + 1] == 1)
        full = full + jnp.where(sel, t1_ref[f:f + 1, :], t0_ref[f:f + 1, :])
    out_a_ref[...] = full[:, :WA]
    out_b_ref[...] = full[:, WA:]


def _make_atom_tc(interpret=False):
    return pl.pallas_call(
        _atom_body,
        interpret=interpret,
        grid=(NB,),
        in_specs=[
            pl.BlockSpec((BR, 16), lambda i: (i, 0)),
            pl.BlockSpec((16, DP), lambda i: (0, 0)),
            pl.BlockSpec((16, DP), lambda i: (0, 0)),
        ],
        out_specs=(pl.BlockSpec((BR, WA), lambda i: (i, 0)),
                   pl.BlockSpec((BR, WB), lambda i: (i, 0))),
        out_shape=(jax.ShapeDtypeStruct((N, WA), _f32),
                   jax.ShapeDtypeStruct((N, WB), _f32)))


BR = 1000
NB = N // BR


def _dot(a, b):
    return jnp.dot(a, b, preferred_element_type=_f32,
                   precision=lax.Precision.HIGHEST)


def _dotd(a, b):
    return jnp.dot(a, b, preferred_element_type=_f32)


def _bn(y, g, b):
    m = jnp.mean(y, axis=0, keepdims=True)
    v = jnp.mean((y - m) * (y - m), axis=0, keepdims=True)
    return (y - m) / jnp.sqrt(v + 1e-5) * g + b


def _layer_mm_body(parts_a_ref, parts_b_ref, stats_ref, wb_ref, wl_ref,
                   cb_ref, y_ref, sums_ref, accm, acc2):
    i = pl.program_id(0)
    nbr = jnp.concatenate(
        [parts_a_ref[0] + parts_a_ref[1],
         parts_b_ref[0] + parts_b_ref[1]], axis=1)
    st = stats_ref[0] + stats_ref[1]
    agg = nbr + _dot(st, wb_ref[...])
    y = _dotd(agg, wl_ref[...]) + cb_ref[...]
    y_ref[...] = y

    @pl.when(i == 0)
    def _():
        accm[...] = jnp.zeros_like(accm)
        acc2[...] = jnp.zeros_like(acc2)

    mb = jnp.mean(y, axis=0, keepdims=True)
    d = y - mb
    accm[pl.ds(i, 1), :] = mb
    acc2[pl.ds(i, 1), :] = jnp.sum(d * d, axis=0, keepdims=True)

    @pl.when(i == NB - 1)
    def _():
        rmask = (lax.broadcasted_iota(jnp.int32, (16, 1), 0) < NB
                 ).astype(_f32)
        means = accm[...] * rmask
        m = jnp.sum(means, axis=0, keepdims=True) * (1.0 / NB)
        dev = (accm[...] - m) * rmask
        v = (jnp.sum(acc2[...] * rmask, axis=0, keepdims=True)
             + BR * jnp.sum(dev * dev, axis=0, keepdims=True)) * (1.0 / N)
        sums_ref[...] = jnp.concatenate(
            [m, v, jnp.zeros((6, DP), _f32)], axis=0)


def _make_layer_mm_tc(interpret=False):
    return pl.pallas_call(
    _layer_mm_body,
    interpret=interpret,
    grid=(NB,),
    in_specs=[
        pl.BlockSpec((NC, BR, WA), lambda i: (0, i, 0)),
        pl.BlockSpec((NC, BR, WB), lambda i: (0, i, 0)),
        pl.BlockSpec((NC, BR, 8), lambda i: (0, i, 0)),
        pl.BlockSpec((8, DP), lambda i: (0, 0)),
        pl.BlockSpec((DP, DP), lambda i: (0, 0)),
        pl.BlockSpec((1, DP), lambda i: (0, 0)),
    ],
    out_specs=(pl.BlockSpec((BR, DP), lambda i: (i, 0)),
               pl.BlockSpec((8, DP), lambda i: (0, 0))),
    out_shape=(jax.ShapeDtypeStruct((N, DP), _f32),
               jax.ShapeDtypeStruct((8, DP), _f32)),
    scratch_shapes=[pltpu.VMEM((16, DP), _f32), pltpu.VMEM((16, DP), _f32)],
    )


def _layer_norm_body(y_ref, sums_ref, g_ref, b_ref, atom_a_ref, atom_b_ref,
                     out_a_ref, out_b_ref):
    m = sums_ref[0:1, :]
    v = sums_ref[1:2, :]
    yn = (y_ref[...] - m) / jnp.sqrt(v + 1e-5) * g_ref[...] + b_ref[...]
    out = jnp.maximum(yn, 0.0)
    out_a_ref[...] = out[:, :WA] + atom_a_ref[...]
    out_b_ref[...] = out[:, WA:] + atom_b_ref[...]


def _make_layer_norm_tc(interpret=False):
    return pl.pallas_call(
    _layer_norm_body,
    interpret=interpret,
    grid=(NB,),
    in_specs=[
        pl.BlockSpec((BR, DP), lambda i: (i, 0)),
        pl.BlockSpec((8, DP), lambda i: (0, 0)),
        pl.BlockSpec((1, DP), lambda i: (0, 0)),
        pl.BlockSpec((1, DP), lambda i: (0, 0)),
        pl.BlockSpec((BR, WA), lambda i: (i, 0)),
        pl.BlockSpec((BR, WB), lambda i: (i, 0)),
    ],
    out_specs=(pl.BlockSpec((BR, WA), lambda i: (i, 0)),
               pl.BlockSpec((BR, WB), lambda i: (i, 0))),
    out_shape=(jax.ShapeDtypeStruct((N, WA), _f32),
               jax.ShapeDtypeStruct((N, WB), _f32)),
    )


def _layer_tc(parts_a, parts_b, stats, atom_a, atom_b, wbl, wll, cbl, gl, bl):
    y, sums = _layer_mm_tc(parts_a, parts_b, stats, wbl, wll, cbl)
    return _layer_norm_tc(y, sums, gl, bl, atom_a, atom_b)


def _pool_body(hx_a_ref, hx_b_ref, batch_ref, wp_ref, pb_ref, prw_ref, prb_ref,
               w1_ref, b1_ref, g1_ref, bb1_ref,
               w2_ref, b2_ref, g2_ref, bb2_ref,
               w3_ref, b3_ref, rw_ref, rb_ref, out_ref, acc):
    i = pl.program_id(0)
    hx = jnp.concatenate([hx_a_ref[...], hx_b_ref[...]], axis=1)
    att = _dotd(hx, wp_ref[...]) + pb_ref[...]
    sc = jax.nn.sigmoid(_dotd(att, prw_ref[...]) + prb_ref[...])
    sh = hx * sc
    gi = lax.broadcasted_iota(jnp.int32, (G, BR), 0)
    oh = (batch_ref[0] == gi).astype(_f32)

    @pl.when(i == 0)
    def _():
        acc[...] = jnp.zeros_like(acc)

    acc[...] += _dot(oh, sh)

    @pl.when(i == NB - 1)
    def _():
        pooled = acc[...]
        z = _dotd(pooled, w1_ref[...]) + b1_ref[...]
        z = jnp.maximum(_bn(z, g1_ref[...], bb1_ref[...]), 0.0)
        z = _dotd(z, w2_ref[...]) + b2_ref[...]
        z = jnp.maximum(_bn(z, g2_ref[...], bb2_ref[...]), 0.0)
        z = _dotd(z, w3_ref[...]) + b3_ref[...]
        out_ref[...] = _dotd(z, rw_ref[...]) + rb_ref[...]


def _w_spec(shape):
    return pl.BlockSpec(shape, lambda i: tuple(0 for _ in shape))


def _make_pool_tc(interpret=False):
    return pl.pallas_call(
    _pool_body,
    interpret=interpret,
    grid=(NB,),
    in_specs=[
        pl.BlockSpec((BR, WA), lambda i: (i, 0)),
        pl.BlockSpec((BR, WB), lambda i: (i, 0)),
        pl.BlockSpec((1, 1, BR), lambda i: (i, 0, 0)),
        _w_spec((DP, DP)), _w_spec((1, DP)), _w_spec((DP, 1)), _w_spec((1, 1)),
        _w_spec((DP, 256)), _w_spec((1, 256)), _w_spec((1, 256)),
        _w_spec((1, 256)),
        _w_spec((256, 128)), _w_spec((1, 128)), _w_spec((1, 128)),
        _w_spec((1, 128)),
        _w_spec((128, P)), _w_spec((1, P)), _w_spec((P, 1)), _w_spec((1, 1)),
    ],
    out_specs=pl.BlockSpec((G, 1), lambda i: (0, 0)),
    out_shape=jax.ShapeDtypeStruct((G, 1), _f32),
    scratch_shapes=[pltpu.VMEM((G, DP), _f32)],
    )


_atom_tc = _make_atom_tc()
_layer_mm_tc = _make_layer_mm_tc()
_layer_norm_tc = _make_layer_norm_tc()
_pool_tc = _make_pool_tc()


def kernel(x, edge_index, edge_attr, batch, atom_emb, bond_emb, conv_A,
           conv_S, conv_b, bn_g, bn_b, pool_A, pool_S, pool_b, pool_rW,
           pool_rb, dn_A1, dn_S1, dn_b1, dn_g1, dn_bb1, dn_A2, dn_S2, dn_b2,
           dn_g2, dn_bb2, dn_A3, dn_S3, dn_b3, dn_rW, dn_rb):
    pad = DP - D
    src = edge_index[0].astype(jnp.int32)
    dst = edge_index[1].astype(jnp.int32)

    vals = jnp.concatenate(
        [jnp.ones((E, 1), _f32), edge_attr.astype(_f32),
         jnp.zeros((E, 4), _f32)], axis=1)

    xi = jnp.pad(x.astype(jnp.int32), ((0, 0), (0, 16 - x.shape[1])))
    t0m = jnp.pad(atom_emb[:, 0, :], ((0, 16 - atom_emb.shape[0]), (0, pad)))
    t1m = jnp.pad(atom_emb[:, 1, :], ((0, 16 - atom_emb.shape[0]), (0, pad)))

    b0 = bond_emb[:, :, 0, :].sum(1)
    dB = bond_emb[:, :, 1, :] - bond_emb[:, :, 0, :]
    wb = jnp.concatenate(
        [b0[:, None, :], dB, jnp.zeros((3, 4, D), _f32)], axis=1)
    wb = jnp.pad(wb, ((0, 0), (0, 0), (0, pad)))

    wl = jnp.stack([_phm_w(conv_A[l], conv_S[l]) for l in range(3)])
    wl = jnp.pad(wl, ((0, 0), (0, pad), (0, pad)))
    cb = jnp.pad(conv_b, ((0, 0), (0, pad)))
    gg = jnp.pad(bn_g, ((0, 0), (0, pad)))
    bb = jnp.pad(bn_b, ((0, 0), (0, pad)))

    zeros_a = jnp.zeros((N, WA), _f32)
    zeros_b = jnp.zeros((N, WB), _f32)
    z8 = jnp.zeros((N, 8), _f32)

    wp = jnp.pad(_phm_w(pool_A, pool_S), ((0, pad), (0, pad)))
    pb = jnp.pad(pool_b, (0, pad)).reshape(1, DP)
    prw = jnp.pad(pool_rW, ((0, pad), (0, 0)))
    prb = pool_rb.reshape(1, 1)
    w1 = jnp.pad(_phm_w(dn_A1, dn_S1), ((0, pad), (0, 0)))
    w2 = _phm_w(dn_A2, dn_S2)
    w3 = _phm_w(dn_A3, dn_S3)

    atom_a, atom_b = _atom_tc(xi, t0m, t1m)
    stats = _stats_sc(dst, vals, z8)

    hx_a, hx_b = atom_a, atom_b
    for l in range(3):
        nbr_a, nbr_b = _segsum_sc(hx_a, hx_b, src, dst, zeros_a, zeros_b)
        hx_a, hx_b = _layer_tc(nbr_a, nbr_b, stats, atom_a, atom_b,
                               wb[l], wl[l],
                               cb[l].reshape(1, DP), gg[l].reshape(1, DP),
                               bb[l].reshape(1, DP))

    return _pool_tc(hx_a, hx_b, batch.astype(jnp.int32).reshape(NB, 1, BR),
                    wp, pb, prw, prb,
                    w1, dn_b1.reshape(1, -1), dn_g1.reshape(1, -1),
                    dn_bb1.reshape(1, -1),
                    w2, dn_b2.reshape(1, -1), dn_g2.reshape(1, -1),
                    dn_bb2.reshape(1, -1),
                    w3, dn_b3.reshape(1, -1), dn_rW, dn_rb.reshape(1, 1))
